# baseline (speedup 1.0000x reference)
import functools

import jax
import jax.numpy as jnp
from jax import lax
from jax.experimental import pallas as pl
from jax.experimental.pallas import tpu as pltpu
from jax.experimental.pallas import tpu_sc as plsc

_NC = 2
_NS = 16
_NW = _NC * _NS

_K = 64


def _ew_tc_body(emb_ref, w_ref, b_ref, o_ref):
    x = jnp.dot(emb_ref[...], w_ref[...], preferred_element_type=jnp.float32)
    x = x[:, 0] + b_ref[0]
    o_ref[...] = 1.0 / (1.0 + jnp.exp(-x))


def _edge_weights(emb_cat, w_var, b_var):
    e = emb_cat.shape[0]
    be = 2048
    grid = pl.cdiv(e, be)
    return pl.pallas_call(
        _ew_tc_body,
        grid=(grid,),
        in_specs=[
            pl.BlockSpec((be, emb_cat.shape[1]), lambda i: (i, 0)),
            pl.BlockSpec((emb_cat.shape[1], 1), lambda i: (0, 0)),
            pl.BlockSpec(memory_space=pltpu.SMEM),
        ],
        out_specs=pl.BlockSpec((be,), lambda i: (i,)),
        out_shape=jax.ShapeDtypeStruct((e,), jnp.float32),
    )(emb_cat, w_var, b_var)


def _deg_sc(col_p, ew_p, n, nb):
    mesh = plsc.VectorSubcoreMesh(core_axis_name="c", subcore_axis_name="s", num_cores=_NC, num_subcores=_NS)

    @functools.partial(
        pl.kernel,
        out_type=[
            jax.ShapeDtypeStruct((n,), jnp.float32),
            jax.ShapeDtypeStruct((n,), jnp.float32),
        ],
        mesh=mesh,
        scratch_types=[
            pltpu.VMEM((2, _K), jnp.int32),
            pltpu.VMEM((2, _K), jnp.float32),
            pltpu.VMEM((1024,), jnp.float32),
            pltpu.VMEM((1000,), jnp.float32),
            pltpu.VMEM_SHARED((n,), jnp.float32),
            pltpu.SemaphoreType.DMA((2,)),
        ],
    )
    def deg_kernel(col_hbm, ew_hbm, out0_hbm, out1_hbm, colb_v, ewf_v, zero_v,
                   bounce_v, acc_sh, ssem):
        c = lax.axis_index("c")
        s = lax.axis_index("s")
        wid = s * _NC + c

        def zfill(i, _):
            zero_v[pl.ds(i * 16, 16)] = jnp.zeros((16,), jnp.float32)
            return 0

        lax.fori_loop(0, 64, zfill, 0)

        @pl.when(s < 10)
        def _():
            pltpu.sync_copy(zero_v.at[pl.ds(0, 1000)], acc_sh.at[pl.ds(s * 1000, 1000)])

        plsc.subcore_barrier()

        blk0 = wid * nb
        pltpu.sync_copy(col_hbm.at[blk0], colb_v.at[0])
        pltpu.sync_copy(ew_hbm.at[blk0], ewf_v.at[0])

        def block(b, _):
            p = lax.rem(b, 2)
            q = lax.rem(b + 1, 2)

            @pl.when(b + 1 < nb)
            def _():
                @pl.when(b >= 1)
                def _():
                    pltpu.make_async_copy(
                        ewf_v.at[q], acc_sh.at[colb_v.at[q]], ssem.at[q]).wait()

                pltpu.sync_copy(col_hbm.at[blk0 + b + 1], colb_v.at[q])
                pltpu.sync_copy(ew_hbm.at[blk0 + b + 1], ewf_v.at[q])

            pltpu.async_copy(ewf_v.at[p], acc_sh.at[colb_v.at[p]], ssem.at[p],
                             add=True)
            return 0

        lax.fori_loop(0, nb, block, 0)
        if nb >= 2:
            pltpu.make_async_copy(ewf_v.at[(nb - 2) % 2],
                                  acc_sh.at[colb_v.at[(nb - 2) % 2]],
                                  ssem.at[(nb - 2) % 2]).wait()
        pltpu.make_async_copy(ewf_v.at[(nb - 1) % 2],
                              acc_sh.at[colb_v.at[(nb - 1) % 2]],
                              ssem.at[(nb - 1) % 2]).wait()
        plsc.subcore_barrier()

        @pl.when(s < 10)
        def _():
            pltpu.sync_copy(acc_sh.at[pl.ds(s * 1000, 1000)], bounce_v)

        @pl.when((s < 10) & (c == 0))
        def _():
            pltpu.sync_copy(bounce_v, out0_hbm.at[pl.ds(s * 1000, 1000)])

        @pl.when((s < 10) & (c == 1))
        def _():
            pltpu.sync_copy(bounce_v, out1_hbm.at[pl.ds(s * 1000, 1000)])

    return deg_kernel(col_p, ew_p)


def _agg_sc(y, row_p, col_p, ew_p, nb):
    n, d = y.shape
    mesh = plsc.VectorSubcoreMesh(core_axis_name="c", subcore_axis_name="s", num_cores=_NC, num_subcores=_NS)

    @functools.partial(
        pl.kernel,
        out_type=[
            jax.ShapeDtypeStruct((n, d), jnp.float32),
            jax.ShapeDtypeStruct((n, d), jnp.float32),
        ],
        mesh=mesh,
        scratch_types=[
            pltpu.VMEM((_K,), jnp.int32),
            pltpu.VMEM((_K,), jnp.int32),
            pltpu.VMEM((_K,), jnp.int32),
            pltpu.VMEM((_K,), jnp.int32),
            pltpu.VMEM((_K,), jnp.float32),
            pltpu.VMEM((_K,), jnp.float32),
            pltpu.VMEM((_K, d), jnp.float32),
            pltpu.VMEM((_K, d), jnp.float32),
            pltpu.VMEM((96, d), jnp.float32),
            pltpu.VMEM_SHARED((n, d), jnp.float32),
            pltpu.SemaphoreType.DMA,
            pltpu.SemaphoreType.DMA,
        ],
    )
    def agg_kernel(y_hbm, row_hbm, col_hbm, ew_hbm, out0_hbm, out1_hbm,
                   row0_v, row1_v, col0_v, col1_v, ew0_v, ew1_v,
                   msg0_v, msg1_v, zero_v, acc_sh, gsem0, gsem1):
        c = lax.axis_index("c")
        s = lax.axis_index("s")
        wid = s * _NC + c
        rows = (row0_v, row1_v)
        cols = (col0_v, col1_v)
        ews = (ew0_v, ew1_v)
        msgs = (msg0_v, msg1_v)
        gsems = (gsem0, gsem1)

        def zfill(i, _):
            r = i // 8
            q = i % 8
            zero_v[r, pl.ds(q * 16, 16)] = jnp.zeros((16,), jnp.float32)
            return 0

        lax.fori_loop(0, 96 * 8, zfill, 0)

        @pl.when(s < 10)
        def _():
            for t in range(10):
                pltpu.sync_copy(zero_v, acc_sh.at[pl.ds(s * 1000 + t * 96, 96)])
            pltpu.sync_copy(zero_v.at[pl.ds(0, 40)],
                            acc_sh.at[pl.ds(s * 1000 + 960, 40)])

        plsc.subcore_barrier()

        blk0 = wid * nb
        pltpu.sync_copy(row_hbm.at[blk0], row0_v)
        pltpu.sync_copy(col_hbm.at[blk0], col0_v)
        pltpu.sync_copy(ew_hbm.at[blk0], ew0_v)
        pltpu.async_copy(y_hbm.at[row0_v], msg0_v, gsem0)

        def pair(gg, _):
            for bb in range(2):
                b = gg * 2 + bb
                nx = 1 - bb

                @pl.when(b + 1 < nb)
                def _(bb=bb, nx=nx, b=b):
                    pltpu.sync_copy(row_hbm.at[blk0 + b + 1], rows[nx])
                    pltpu.sync_copy(col_hbm.at[blk0 + b + 1], cols[nx])
                    pltpu.sync_copy(ew_hbm.at[blk0 + b + 1], ews[nx])
                    pltpu.async_copy(y_hbm.at[rows[nx]], msgs[nx], gsems[nx])

                pltpu.make_async_copy(
                    y_hbm.at[rows[bb]], msgs[bb], gsems[bb]).wait()

                def scale16(g, _, bb=bb):
                    ewv = ews[bb][pl.ds(g * 16, 16)]
                    mv = msgs[bb]
                    for jj in range(16):
                        bc = ewv[jj]
                        j = g * 16 + jj
                        for q2 in range(d // 16):
                            mv[j, pl.ds(q2 * 16, 16)] = (
                                mv[j, pl.ds(q2 * 16, 16)] * bc)
                    return 0

                lax.fori_loop(0, _K // 16, scale16, 0)
                pltpu.sync_copy(msgs[bb], acc_sh.at[cols[bb]], add=True)
            return 0

        lax.fori_loop(0, nb // 2, pair, 0)
        plsc.subcore_barrier()

        @pl.when(s < 10)
        def _():
            for t in range(8):
                r = 128 if t < 7 else 104
                base = s * 1000 + t * 128
                pltpu.sync_copy(acc_sh.at[pl.ds(base, r)],
                                msg0_v.at[pl.ds(0, r)])

                @pl.when(c == 0)
                def _(r=r, base=base):
                    pltpu.sync_copy(msg0_v.at[pl.ds(0, r)],
                                    out0_hbm.at[pl.ds(base, r)])

                @pl.when(c == 1)
                def _(r=r, base=base):
                    pltpu.sync_copy(msg0_v.at[pl.ds(0, r)],
                                    out1_hbm.at[pl.ds(base, r)])

    return agg_kernel(y, row_p, col_p, ew_p)


def _y_first_body(deg0_ref, deg1_ref, feat_ref, w_ref, dinv_ref, y_ref):
    deg = 1.0 + deg0_ref[...] + deg1_ref[...]
    dinv = lax.rsqrt(deg)
    dinv_ref[...] = dinv
    xw = jnp.dot(feat_ref[...], w_ref[...], preferred_element_type=jnp.float32)
    y_ref[...] = xw * dinv[:, None]


def _y_first(deg0, deg1, features, w1):
    n, din = features.shape
    hid = w1.shape[1]
    bn = 2048
    grid = pl.cdiv(n, bn)
    return pl.pallas_call(
        _y_first_body,
        grid=(grid,),
        in_specs=[
            pl.BlockSpec((bn,), lambda i: (i,)),
            pl.BlockSpec((bn,), lambda i: (i,)),
            pl.BlockSpec((bn, din), lambda i: (i, 0)),
            pl.BlockSpec((din, hid), lambda i: (0, 0)),
        ],
        out_specs=[
            pl.BlockSpec((bn,), lambda i: (i,)),
            pl.BlockSpec((bn, hid), lambda i: (i, 0)),
        ],
        out_shape=[
            jax.ShapeDtypeStruct((n,), jnp.float32),
            jax.ShapeDtypeStruct((n, hid), jnp.float32),
        ],
    )(deg0, deg1, features, w1)


def _y_second_body(a0_ref, a1_ref, y_ref, dinv_ref, b_ref, w_ref, y2_ref):
    dinv = dinv_ref[...]
    h = dinv[:, None] * (a0_ref[...] + a1_ref[...] + y_ref[...]) + b_ref[...][None, :]
    h = jnp.maximum(h, 0.0)
    xw = jnp.dot(h, w_ref[...], preferred_element_type=jnp.float32)
    y2_ref[...] = xw * dinv[:, None]


def _y_second(acc0, acc1, y1, dinv, b1, w2):
    n, hid = y1.shape
    bn = 2048
    grid = pl.cdiv(n, bn)
    return pl.pallas_call(
        _y_second_body,
        grid=(grid,),
        in_specs=[
            pl.BlockSpec((bn, hid), lambda i: (i, 0)),
            pl.BlockSpec((bn, hid), lambda i: (i, 0)),
            pl.BlockSpec((bn, hid), lambda i: (i, 0)),
            pl.BlockSpec((bn,), lambda i: (i,)),
            pl.BlockSpec((hid,), lambda i: (0,)),
            pl.BlockSpec((hid, hid), lambda i: (0, 0)),
        ],
        out_specs=pl.BlockSpec((bn, hid), lambda i: (i, 0)),
        out_shape=jax.ShapeDtypeStruct((n, hid), jnp.float32),
    )(acc0, acc1, y1, dinv, b1, w2)


def _final_body(a0_ref, a1_ref, y2_ref, dinv_ref, b2_ref, lab_ref, wd_ref,
                sc_ref, o_ref):
    dinv = dinv_ref[...]
    h = dinv[:, None] * (a0_ref[...] + a1_ref[...] + y2_ref[...]) + b2_ref[...][None, :]
    delta = jnp.dot(h, wd_ref[...], preferred_element_type=jnp.float32)[:, 0]
    delta = delta + lab_ref[...] * sc_ref[0] + sc_ref[1]
    p1 = 1.0 / (1.0 + jnp.exp(-delta))
    o_ref[...] = jnp.concatenate([(1.0 - p1)[:, None], p1[:, None]], axis=1)


def _final(acc0, acc1, y2, dinv, b2, labels_f, cls_w, cls_b):
    n, hid = y2.shape
    bn = 2048
    grid = pl.cdiv(n, bn)
    wd = cls_w[:hid, 1:2] - cls_w[:hid, 0:1]
    sc = jnp.stack([cls_w[hid, 1] - cls_w[hid, 0], cls_b[1] - cls_b[0]])
    return pl.pallas_call(
        _final_body,
        grid=(grid,),
        in_specs=[
            pl.BlockSpec((bn, hid), lambda i: (i, 0)),
            pl.BlockSpec((bn, hid), lambda i: (i, 0)),
            pl.BlockSpec((bn, hid), lambda i: (i, 0)),
            pl.BlockSpec((bn,), lambda i: (i,)),
            pl.BlockSpec((hid,), lambda i: (0,)),
            pl.BlockSpec((bn,), lambda i: (i,)),
            pl.BlockSpec((hid, 1), lambda i: (0, 0)),
            pl.BlockSpec(memory_space=pltpu.SMEM),
        ],
        out_specs=pl.BlockSpec((bn, 2), lambda i: (i, 0)),
        out_shape=jax.ShapeDtypeStruct((n, 2), jnp.float32),
    )(acc0, acc1, y2, dinv, b2, labels_f, wd, sc)


def kernel(emb_cat, features, edge_index, labels, W_var, b_var,
           gcn1_W, gcn1_b, gcn2_W, gcn2_b, cls_W, cls_b):
    n = features.shape[0]
    e = edge_index.shape[1]
    row = edge_index[0].astype(jnp.int32)
    col = edge_index[1].astype(jnp.int32)

    ew = _edge_weights(emb_cat, W_var, b_var)

    per_w = e // _NW
    nb = pl.cdiv(per_w, _K)
    nb = nb + (nb % 2)
    pad = nb * _K - per_w
    padw = ((0, 0), (0, pad))
    row_p = jnp.pad(row.reshape(_NW, per_w), padw).reshape(_NW * nb, _K)
    col_p = jnp.pad(col.reshape(_NW, per_w), padw).reshape(_NW * nb, _K)
    ew_p = jnp.pad(ew.reshape(_NW, per_w), padw).reshape(_NW * nb, _K)

    deg0, deg1 = _deg_sc(col_p, ew_p, n, nb)
    dinv, y1 = _y_first(deg0, deg1, features, gcn1_W)
    acc1_0, acc1_1 = _agg_sc(y1, row_p, col_p, ew_p, nb)
    y2 = _y_second(acc1_0, acc1_1, y1, dinv, gcn1_b, gcn2_W)
    acc2_0, acc2_1 = _agg_sc(y2, row_p, col_p, ew_p, nb)
    return _final(acc2_0, acc2_1, y2, dinv, gcn2_b,
                  labels.astype(jnp.float32), cls_W, cls_b)


# speedup vs baseline: 1.1614x; 1.1614x over previous
import functools

import jax
import jax.numpy as jnp
from jax import lax
from jax.experimental import pallas as pl
from jax.experimental.pallas import tpu as pltpu
from jax.experimental.pallas import tpu_sc as plsc

_NC = 2
_NS = 16
_NW = _NC * _NS

_K = 80


def _ew_tc_body(emb_ref, w_ref, b_ref, o_ref):
    x = jnp.dot(emb_ref[...], w_ref[...], preferred_element_type=jnp.float32)
    x = x[:, 0] + b_ref[0]
    o_ref[...] = 1.0 / (1.0 + jnp.exp(-x))


def _edge_weights(emb_cat, w_var, b_var):
    e = emb_cat.shape[0]
    be = 2048
    grid = pl.cdiv(e, be)
    return pl.pallas_call(
        _ew_tc_body,
        grid=(grid,),
        in_specs=[
            pl.BlockSpec((be, emb_cat.shape[1]), lambda i: (i, 0)),
            pl.BlockSpec((emb_cat.shape[1], 1), lambda i: (0, 0)),
            pl.BlockSpec(memory_space=pltpu.SMEM),
        ],
        out_specs=pl.BlockSpec((be,), lambda i: (i,)),
        out_shape=jax.ShapeDtypeStruct((e,), jnp.float32),
    )(emb_cat, w_var, b_var)


def _deg_sc(col_p, ew_p, n, nb):
    mesh = plsc.VectorSubcoreMesh(core_axis_name="c", subcore_axis_name="s", num_cores=_NC, num_subcores=_NS)

    @functools.partial(
        pl.kernel,
        out_type=[
            jax.ShapeDtypeStruct((n,), jnp.float32),
            jax.ShapeDtypeStruct((n,), jnp.float32),
        ],
        mesh=mesh,
        scratch_types=[
            pltpu.VMEM((2, _K), jnp.int32),
            pltpu.VMEM((2, _K), jnp.float32),
            pltpu.VMEM((1024,), jnp.float32),
            pltpu.VMEM((1000,), jnp.float32),
            pltpu.VMEM_SHARED((n,), jnp.float32),
            pltpu.SemaphoreType.DMA((2,)),
        ],
    )
    def deg_kernel(col_hbm, ew_hbm, out0_hbm, out1_hbm, colb_v, ewf_v, zero_v,
                   bounce_v, acc_sh, ssem):
        c = lax.axis_index("c")
        s = lax.axis_index("s")
        wid = s * _NC + c

        def zfill(i, _):
            zero_v[pl.ds(i * 16, 16)] = jnp.zeros((16,), jnp.float32)
            return 0

        lax.fori_loop(0, 64, zfill, 0)

        @pl.when(s < 10)
        def _():
            pltpu.sync_copy(zero_v.at[pl.ds(0, 1000)], acc_sh.at[pl.ds(s * 1000, 1000)])

        plsc.subcore_barrier()

        blk0 = wid * nb
        pltpu.sync_copy(col_hbm.at[blk0], colb_v.at[0])
        pltpu.sync_copy(ew_hbm.at[blk0], ewf_v.at[0])

        def block(b, _):
            p = lax.rem(b, 2)
            q = lax.rem(b + 1, 2)

            @pl.when(b + 1 < nb)
            def _():
                @pl.when(b >= 1)
                def _():
                    pltpu.make_async_copy(
                        ewf_v.at[q], acc_sh.at[colb_v.at[q]], ssem.at[q]).wait()

                pltpu.sync_copy(col_hbm.at[blk0 + b + 1], colb_v.at[q])
                pltpu.sync_copy(ew_hbm.at[blk0 + b + 1], ewf_v.at[q])

            pltpu.async_copy(ewf_v.at[p], acc_sh.at[colb_v.at[p]], ssem.at[p],
                             add=True)
            return 0

        lax.fori_loop(0, nb, block, 0)
        if nb >= 2:
            pltpu.make_async_copy(ewf_v.at[(nb - 2) % 2],
                                  acc_sh.at[colb_v.at[(nb - 2) % 2]],
                                  ssem.at[(nb - 2) % 2]).wait()
        pltpu.make_async_copy(ewf_v.at[(nb - 1) % 2],
                              acc_sh.at[colb_v.at[(nb - 1) % 2]],
                              ssem.at[(nb - 1) % 2]).wait()
        plsc.subcore_barrier()

        @pl.when(s < 10)
        def _():
            pltpu.sync_copy(acc_sh.at[pl.ds(s * 1000, 1000)], bounce_v)

        @pl.when((s < 10) & (c == 0))
        def _():
            pltpu.sync_copy(bounce_v, out0_hbm.at[pl.ds(s * 1000, 1000)])

        @pl.when((s < 10) & (c == 1))
        def _():
            pltpu.sync_copy(bounce_v, out1_hbm.at[pl.ds(s * 1000, 1000)])

    return deg_kernel(col_p, ew_p)


def _agg_sc(y, row_p, col_p, ew_p, nb):
    n, d = y.shape
    mesh = plsc.VectorSubcoreMesh(core_axis_name="c", subcore_axis_name="s", num_cores=_NC, num_subcores=_NS)

    @functools.partial(
        pl.kernel,
        out_type=[
            jax.ShapeDtypeStruct((n, d), jnp.float32),
            jax.ShapeDtypeStruct((n, d), jnp.float32),
        ],
        mesh=mesh,
        scratch_types=[
            pltpu.VMEM((_K,), jnp.int32),
            pltpu.VMEM((_K,), jnp.int32),
            pltpu.VMEM((_K,), jnp.int32),
            pltpu.VMEM((_K,), jnp.int32),
            pltpu.VMEM((_K,), jnp.float32),
            pltpu.VMEM((_K,), jnp.float32),
            pltpu.VMEM((_K, d), jnp.float32),
            pltpu.VMEM((_K, d), jnp.float32),
            pltpu.VMEM((96, d), jnp.float32),
            pltpu.VMEM_SHARED((n, d), jnp.float32),
            pltpu.SemaphoreType.DMA,
            pltpu.SemaphoreType.DMA,
        ],
    )
    def agg_kernel(y_hbm, row_hbm, col_hbm, ew_hbm, out0_hbm, out1_hbm,
                   row0_v, row1_v, col0_v, col1_v, ew0_v, ew1_v,
                   msg0_v, msg1_v, zero_v, acc_sh, gsem0, gsem1):
        c = lax.axis_index("c")
        s = lax.axis_index("s")
        wid = s * _NC + c
        rows = (row0_v, row1_v)
        cols = (col0_v, col1_v)
        ews = (ew0_v, ew1_v)
        msgs = (msg0_v, msg1_v)
        gsems = (gsem0, gsem1)

        def zfill(i, _):
            r = i // 8
            q = i % 8
            zero_v[r, pl.ds(q * 16, 16)] = jnp.zeros((16,), jnp.float32)
            return 0

        lax.fori_loop(0, 96 * 8, zfill, 0)

        @pl.when(s < 10)
        def _():
            for t in range(10):
                pltpu.sync_copy(zero_v, acc_sh.at[pl.ds(s * 1000 + t * 96, 96)])
            pltpu.sync_copy(zero_v.at[pl.ds(0, 40)],
                            acc_sh.at[pl.ds(s * 1000 + 960, 40)])

        plsc.subcore_barrier()

        blk0 = wid * nb
        pltpu.sync_copy(row_hbm.at[blk0], row0_v)
        pltpu.sync_copy(col_hbm.at[blk0], col0_v)
        pltpu.sync_copy(ew_hbm.at[blk0], ew0_v)
        pltpu.async_copy(y_hbm.at[row0_v], msg0_v, gsem0)

        def pair(gg, _):
            for bb in range(2):
                b = gg * 2 + bb
                nx = 1 - bb

                @pl.when(b + 1 < nb)
                def _(bb=bb, nx=nx, b=b):
                    pltpu.sync_copy(row_hbm.at[blk0 + b + 1], rows[nx])
                    pltpu.sync_copy(col_hbm.at[blk0 + b + 1], cols[nx])
                    pltpu.sync_copy(ew_hbm.at[blk0 + b + 1], ews[nx])
                    pltpu.async_copy(y_hbm.at[rows[nx]], msgs[nx], gsems[nx])

                pltpu.make_async_copy(
                    y_hbm.at[rows[bb]], msgs[bb], gsems[bb]).wait()

                def scale16(g, _, bb=bb):
                    ewv = ews[bb][pl.ds(g * 16, 16)]
                    mv = msgs[bb]
                    for jj in range(16):
                        bc = ewv[jj]
                        j = g * 16 + jj
                        for q2 in range(d // 16):
                            mv[j, pl.ds(q2 * 16, 16)] = (
                                mv[j, pl.ds(q2 * 16, 16)] * bc)
                    return 0

                lax.fori_loop(0, _K // 16, scale16, 0)
                pltpu.sync_copy(msgs[bb], acc_sh.at[cols[bb]], add=True)
            return 0

        lax.fori_loop(0, nb // 2, pair, 0)
        plsc.subcore_barrier()

        @pl.when(s < 10)
        def _():
            for t in range(8):
                r = 128 if t < 7 else 104
                base = s * 1000 + t * 128
                pltpu.sync_copy(acc_sh.at[pl.ds(base, r)],
                                msg0_v.at[pl.ds(0, r)])

                @pl.when(c == 0)
                def _(r=r, base=base):
                    pltpu.sync_copy(msg0_v.at[pl.ds(0, r)],
                                    out0_hbm.at[pl.ds(base, r)])

                @pl.when(c == 1)
                def _(r=r, base=base):
                    pltpu.sync_copy(msg0_v.at[pl.ds(0, r)],
                                    out1_hbm.at[pl.ds(base, r)])

    return agg_kernel(y, row_p, col_p, ew_p)


def _y_first_body(deg0_ref, deg1_ref, feat_ref, w_ref, dinv_ref, y_ref):
    deg = 1.0 + deg0_ref[...] + deg1_ref[...]
    dinv = lax.rsqrt(deg)
    dinv_ref[...] = dinv
    xw = jnp.dot(feat_ref[...], w_ref[...], preferred_element_type=jnp.float32)
    y_ref[...] = xw * dinv[:, None]


def _y_first(deg0, deg1, features, w1):
    n, din = features.shape
    hid = w1.shape[1]
    bn = 2048
    grid = pl.cdiv(n, bn)
    return pl.pallas_call(
        _y_first_body,
        grid=(grid,),
        in_specs=[
            pl.BlockSpec((bn,), lambda i: (i,)),
            pl.BlockSpec((bn,), lambda i: (i,)),
            pl.BlockSpec((bn, din), lambda i: (i, 0)),
            pl.BlockSpec((din, hid), lambda i: (0, 0)),
        ],
        out_specs=[
            pl.BlockSpec((bn,), lambda i: (i,)),
            pl.BlockSpec((bn, hid), lambda i: (i, 0)),
        ],
        out_shape=[
            jax.ShapeDtypeStruct((n,), jnp.float32),
            jax.ShapeDtypeStruct((n, hid), jnp.float32),
        ],
    )(deg0, deg1, features, w1)


def _y_second_body(a0_ref, a1_ref, y_ref, dinv_ref, b_ref, w_ref, y2_ref):
    dinv = dinv_ref[...]
    h = dinv[:, None] * (a0_ref[...] + a1_ref[...] + y_ref[...]) + b_ref[...][None, :]
    h = jnp.maximum(h, 0.0)
    xw = jnp.dot(h, w_ref[...], preferred_element_type=jnp.float32)
    y2_ref[...] = xw * dinv[:, None]


def _y_second(acc0, acc1, y1, dinv, b1, w2):
    n, hid = y1.shape
    bn = 2048
    grid = pl.cdiv(n, bn)
    return pl.pallas_call(
        _y_second_body,
        grid=(grid,),
        in_specs=[
            pl.BlockSpec((bn, hid), lambda i: (i, 0)),
            pl.BlockSpec((bn, hid), lambda i: (i, 0)),
            pl.BlockSpec((bn, hid), lambda i: (i, 0)),
            pl.BlockSpec((bn,), lambda i: (i,)),
            pl.BlockSpec((hid,), lambda i: (0,)),
            pl.BlockSpec((hid, hid), lambda i: (0, 0)),
        ],
        out_specs=pl.BlockSpec((bn, hid), lambda i: (i, 0)),
        out_shape=jax.ShapeDtypeStruct((n, hid), jnp.float32),
    )(acc0, acc1, y1, dinv, b1, w2)


def _final_body(a0_ref, a1_ref, y2_ref, dinv_ref, b2_ref, lab_ref, wd_ref,
                sc_ref, o_ref):
    dinv = dinv_ref[...]
    h = dinv[:, None] * (a0_ref[...] + a1_ref[...] + y2_ref[...]) + b2_ref[...][None, :]
    delta = jnp.dot(h, wd_ref[...], preferred_element_type=jnp.float32)[:, 0]
    delta = delta + lab_ref[...] * sc_ref[0] + sc_ref[1]
    p1 = 1.0 / (1.0 + jnp.exp(-delta))
    o_ref[...] = jnp.concatenate([(1.0 - p1)[:, None], p1[:, None]], axis=1)


def _final(acc0, acc1, y2, dinv, b2, labels_f, cls_w, cls_b):
    n, hid = y2.shape
    bn = 2048
    grid = pl.cdiv(n, bn)
    wd = cls_w[:hid, 1:2] - cls_w[:hid, 0:1]
    sc = jnp.stack([cls_w[hid, 1] - cls_w[hid, 0], cls_b[1] - cls_b[0]])
    return pl.pallas_call(
        _final_body,
        grid=(grid,),
        in_specs=[
            pl.BlockSpec((bn, hid), lambda i: (i, 0)),
            pl.BlockSpec((bn, hid), lambda i: (i, 0)),
            pl.BlockSpec((bn, hid), lambda i: (i, 0)),
            pl.BlockSpec((bn,), lambda i: (i,)),
            pl.BlockSpec((hid,), lambda i: (0,)),
            pl.BlockSpec((bn,), lambda i: (i,)),
            pl.BlockSpec((hid, 1), lambda i: (0, 0)),
            pl.BlockSpec(memory_space=pltpu.SMEM),
        ],
        out_specs=pl.BlockSpec((bn, 2), lambda i: (i, 0)),
        out_shape=jax.ShapeDtypeStruct((n, 2), jnp.float32),
    )(acc0, acc1, y2, dinv, b2, labels_f, wd, sc)


def kernel(emb_cat, features, edge_index, labels, W_var, b_var,
           gcn1_W, gcn1_b, gcn2_W, gcn2_b, cls_W, cls_b):
    n = features.shape[0]
    e = edge_index.shape[1]
    row = edge_index[0].astype(jnp.int32)
    col = edge_index[1].astype(jnp.int32)

    ew = _edge_weights(emb_cat, W_var, b_var)

    per_w = e // _NW
    nb = pl.cdiv(per_w, _K)
    nb = nb + (nb % 2)
    pad = nb * _K - per_w
    padw = ((0, 0), (0, pad))
    row_p = jnp.pad(row.reshape(_NW, per_w), padw).reshape(_NW * nb, _K)
    col_p = jnp.pad(col.reshape(_NW, per_w), padw).reshape(_NW * nb, _K)
    ew_p = jnp.pad(ew.reshape(_NW, per_w), padw).reshape(_NW * nb, _K)

    deg0, deg1 = _deg_sc(col_p, ew_p, n, nb)
    dinv, y1 = _y_first(deg0, deg1, features, gcn1_W)
    acc1_0, acc1_1 = _agg_sc(y1, row_p, col_p, ew_p, nb)
    y2 = _y_second(acc1_0, acc1_1, y1, dinv, gcn1_b, gcn2_W)
    acc2_0, acc2_1 = _agg_sc(y2, row_p, col_p, ew_p, nb)
    return _final(acc2_0, acc2_1, y2, dinv, gcn2_b,
                  labels.astype(jnp.float32), cls_W, cls_b)
